# baseline (speedup 1.0000x reference)
import functools

import jax
import jax.numpy as jnp
from jax import lax
from jax.experimental import pallas as pl
from jax.experimental.pallas import tpu as pltpu
from jax.experimental.pallas import tpu_sc as plsc

_B = 16
_N = 3125
_M = 9375
_NNZ = 37500
_D = 128
_S = 3

_L = 16
_TPG = 2
_E = 18752
_MP = 9376
_STRIDE = 3200
_WROW = _S * _STRIDE


def _sc_body(rows_hbm, cols_hbm, vals_hbm, didx_hbm, w_hbm,
             rows_v, cols_v, vals_v, didx_v, w_v, sem):
    cid = lax.axis_index("c")
    sid = lax.axis_index("s")
    wid = sid * 2 + cid
    g = wid // _TPG

    h = wid % _TPG
    e0 = h * _E
    d0 = pltpu.async_copy(rows_hbm.at[wid], rows_v, sem)
    d1 = pltpu.async_copy(cols_hbm.at[wid], cols_v, sem)
    d2 = pltpu.async_copy(vals_hbm.at[wid], vals_v, sem)
    d3 = pltpu.async_copy(didx_hbm.at[g], didx_v, sem)

    zeros = jnp.zeros((_L,), jnp.float32)

    for row in range(_S):
        @plsc.parallel_loop(0, _STRIDE // _L, 1, unroll=8)
        def zero_body(j, row=row):
            w_v[row, pl.ds(j * _L, _L)] = zeros

    d0.wait()
    d1.wait()
    d2.wait()
    d3.wait()

    shift = lax.rem(g * _N, 8)

    @plsc.parallel_loop(0, _E // _L, 1, unroll=8)
    def edge_body(j):
        off = j * _L
        r = rows_v[pl.ds(off, _L)]
        c = cols_v[pl.ds(off, _L)]
        v = vals_v[pl.ds(off, _L)]
        s = plsc.load_gather(didx_v, [r])
        plsc.addupdate_scatter(w_v, [s, c + shift], v)

    pltpu.sync_copy(w_v, w_hbm.at[wid])


_sc_kernel = pl.kernel(
    _sc_body,
    out_type=jax.ShapeDtypeStruct((_B * _TPG, _S, _STRIDE), jnp.float32),
    mesh=plsc.VectorSubcoreMesh(core_axis_name="c", subcore_axis_name="s"),
    scratch_types=[
        pltpu.VMEM((_E,), jnp.int32),
        pltpu.VMEM((_E,), jnp.int32),
        pltpu.VMEM((_E,), jnp.float32),
        pltpu.VMEM((_MP,), jnp.int32),
        pltpu.VMEM((_S, _STRIDE), jnp.float32),
        pltpu.SemaphoreType.DMA,
    ],
    compiler_params=pltpu.CompilerParams(needs_layout_passes=False),
)


_SLAB = _STRIDE
_SLAB_LAST = 3128


def _slab_dma(x_hbm, xbuf, sem, b, buf):
    s8 = pl.multiple_of((b * _N // 8) * 8, 8)

    def _full():
        return pltpu.make_async_copy(
            x_hbm.at[pl.ds(s8, _SLAB), :], xbuf.at[buf], sem.at[buf])

    def _last():
        return pltpu.make_async_copy(
            x_hbm.at[pl.ds(s8, _SLAB_LAST), :],
            xbuf.at[buf, pl.ds(0, _SLAB_LAST)], sem.at[buf])

    return _full, _last


_NBUF = 4


def _start_slab(x_hbm, xbuf, sem, b, buf):
    full, last = _slab_dma(x_hbm, xbuf, sem, b, buf)

    @pl.when(b < _B - 1)
    def _():
        full().start()

    @pl.when(b == _B - 1)
    def _():
        last().start()


def _tc_body(wp_ref, x_hbm, out_ref, xbuf, sem):
    b = pl.program_id(0)
    cur = lax.rem(b, _NBUF)

    @pl.when(b == 0)
    def _():
        for k in range(_NBUF - 1):
            _start_slab(x_hbm, xbuf, sem, k, k)

    @pl.when(b + _NBUF - 1 < _B)
    def _():
        _start_slab(x_hbm, xbuf, sem, b + _NBUF - 1,
                    lax.rem(b + _NBUF - 1, _NBUF))

    full, last = _slab_dma(x_hbm, xbuf, sem, b, cur)

    @pl.when(b < _B - 1)
    def _():
        full().wait()

    @pl.when(b == _B - 1)
    def _():
        last().wait()

    w = wp_ref[0] + wp_ref[1]
    out_ref[0] = jnp.dot(w, xbuf[cur], preferred_element_type=jnp.float32)


_tc_matmul = pl.pallas_call(
    _tc_body,
    grid=(_B,),
    in_specs=[
        pl.BlockSpec((_TPG, _S, _STRIDE), lambda b: (b, 0, 0)),
        pl.BlockSpec(memory_space=pl.ANY),
    ],
    out_specs=pl.BlockSpec((1, _S, _D), lambda b: (b, 0, 0)),
    out_shape=jax.ShapeDtypeStruct((_B, _S, _D), jnp.float32),
    scratch_shapes=[
        pltpu.VMEM((_NBUF, _SLAB, _D), jnp.float32),
        pltpu.SemaphoreType.DMA((_NBUF,)),
    ],
)


def kernel(x, batch, batch_size, d_rows, d_cols, d_vals, d_index):
    pad_e = _TPG * _E - _NNZ
    rows_p = jnp.pad(d_rows, ((0, 0), (0, pad_e))).reshape(_B * _TPG, _E)
    cols_p = jnp.pad(d_cols, ((0, 0), (0, pad_e))).reshape(_B * _TPG, _E)
    vals_p = jnp.pad(d_vals, ((0, 0), (0, pad_e))).reshape(_B * _TPG, _E)
    didx_p = jnp.pad(d_index, ((0, 0), (0, _MP - _M)))

    wp = _sc_kernel(rows_p, cols_p, vals_p, didx_p)
    out3 = _tc_matmul(wp, x)
    return out3.reshape(_B, _S * _D)


# speedup vs baseline: 1.0193x; 1.0193x over previous
import functools

import jax
import jax.numpy as jnp
from jax import lax
from jax.experimental import pallas as pl
from jax.experimental.pallas import tpu as pltpu
from jax.experimental.pallas import tpu_sc as plsc

_B = 16
_N = 3125
_M = 9375
_NNZ = 37500
_D = 128
_S = 3

_L = 16
_TPG = 2
_E = 18752
_MP = 9376
_STRIDE = 3200
_WROW = _S * _STRIDE


def _sc_body(rows_hbm, cols_hbm, vals_hbm, didx_hbm, w_hbm,
             rows_v, cols_v, vals_v, didx_v, w_v, sem):
    cid = lax.axis_index("c")
    sid = lax.axis_index("s")
    wid = sid * 2 + cid
    g = wid // _TPG

    h = wid % _TPG
    e0 = h * _E
    d0 = pltpu.async_copy(rows_hbm.at[wid], rows_v, sem)
    d1 = pltpu.async_copy(cols_hbm.at[wid], cols_v, sem)
    d2 = pltpu.async_copy(vals_hbm.at[wid], vals_v, sem)
    d3 = pltpu.async_copy(didx_hbm.at[g], didx_v, sem)

    zeros = jnp.zeros((_L,), jnp.float32)

    for row in range(_S):
        @plsc.parallel_loop(0, _STRIDE // _L, 1, unroll=8)
        def zero_body(j, row=row):
            w_v[row, pl.ds(j * _L, _L)] = zeros

    d3.wait()

    zi = jnp.zeros((_L,), jnp.int32)

    def cnt_body(j, carry):
        a0, a1 = carry
        dv = didx_v[pl.ds(j * _L, _L)]
        a0 = a0 + (dv == 0).astype(jnp.int32)
        a1 = a1 + (dv <= 1).astype(jnp.int32)
        return a0, a1

    a0, a1 = lax.fori_loop(0, _MP // _L, cnt_body, (zi, zi))
    b1 = jnp.sum(a0)
    b2 = jnp.sum(a1)

    d0.wait()
    d1.wait()
    d2.wait()

    shift = lax.rem(g * _N, 8)

    @plsc.parallel_loop(0, _E // _L, 1, unroll=4)
    def edge_body(j):
        off = j * _L
        r = rows_v[pl.ds(off, _L)]
        c = cols_v[pl.ds(off, _L)]
        v = vals_v[pl.ds(off, _L)]
        s = (r >= b1).astype(jnp.int32) + (r >= b2).astype(jnp.int32)
        plsc.addupdate_scatter(w_v, [s, c + shift], v)

    pltpu.sync_copy(w_v, w_hbm.at[wid])


_sc_kernel = pl.kernel(
    _sc_body,
    out_type=jax.ShapeDtypeStruct((_B * _TPG, _S, _STRIDE), jnp.float32),
    mesh=plsc.VectorSubcoreMesh(core_axis_name="c", subcore_axis_name="s"),
    scratch_types=[
        pltpu.VMEM((_E,), jnp.int32),
        pltpu.VMEM((_E,), jnp.int32),
        pltpu.VMEM((_E,), jnp.float32),
        pltpu.VMEM((_MP,), jnp.int32),
        pltpu.VMEM((_S, _STRIDE), jnp.float32),
        pltpu.SemaphoreType.DMA,
    ],
    compiler_params=pltpu.CompilerParams(needs_layout_passes=False),
)


_SLAB = _STRIDE
_SLAB_LAST = 3128


def _slab_dma(x_hbm, xbuf, sem, b, buf):
    s8 = pl.multiple_of((b * _N // 8) * 8, 8)

    def _full():
        return pltpu.make_async_copy(
            x_hbm.at[pl.ds(s8, _SLAB), :], xbuf.at[buf], sem.at[buf])

    def _last():
        return pltpu.make_async_copy(
            x_hbm.at[pl.ds(s8, _SLAB_LAST), :],
            xbuf.at[buf, pl.ds(0, _SLAB_LAST)], sem.at[buf])

    return _full, _last


_NBUF = 4


def _start_slab(x_hbm, xbuf, sem, b, buf):
    full, last = _slab_dma(x_hbm, xbuf, sem, b, buf)

    @pl.when(b < _B - 1)
    def _():
        full().start()

    @pl.when(b == _B - 1)
    def _():
        last().start()


def _tc_body(wp_ref, x_hbm, out_ref, xbuf, sem):
    b = pl.program_id(0)
    cur = lax.rem(b, _NBUF)

    @pl.when(b == 0)
    def _():
        for k in range(_NBUF - 1):
            _start_slab(x_hbm, xbuf, sem, k, k)

    @pl.when(b + _NBUF - 1 < _B)
    def _():
        _start_slab(x_hbm, xbuf, sem, b + _NBUF - 1,
                    lax.rem(b + _NBUF - 1, _NBUF))

    full, last = _slab_dma(x_hbm, xbuf, sem, b, cur)

    @pl.when(b < _B - 1)
    def _():
        full().wait()

    @pl.when(b == _B - 1)
    def _():
        last().wait()

    w = wp_ref[0] + wp_ref[1]
    out_ref[0] = jnp.dot(w, xbuf[cur], preferred_element_type=jnp.float32)


_tc_matmul = pl.pallas_call(
    _tc_body,
    grid=(_B,),
    in_specs=[
        pl.BlockSpec((_TPG, _S, _STRIDE), lambda b: (b, 0, 0)),
        pl.BlockSpec(memory_space=pl.ANY),
    ],
    out_specs=pl.BlockSpec((1, _S, _D), lambda b: (b, 0, 0)),
    out_shape=jax.ShapeDtypeStruct((_B, _S, _D), jnp.float32),
    scratch_shapes=[
        pltpu.VMEM((_NBUF, _SLAB, _D), jnp.float32),
        pltpu.SemaphoreType.DMA((_NBUF,)),
    ],
)


def kernel(x, batch, batch_size, d_rows, d_cols, d_vals, d_index):
    pad_e = _TPG * _E - _NNZ
    rows_p = jnp.pad(d_rows, ((0, 0), (0, pad_e))).reshape(_B * _TPG, _E)
    cols_p = jnp.pad(d_cols, ((0, 0), (0, pad_e))).reshape(_B * _TPG, _E)
    vals_p = jnp.pad(d_vals, ((0, 0), (0, pad_e))).reshape(_B * _TPG, _E)
    didx_p = jnp.pad(d_index, ((0, 0), (0, _MP - _M)), constant_values=2)

    wp = _sc_kernel(rows_p, cols_p, vals_p, didx_p)
    out3 = _tc_matmul(wp, x)
    return out3.reshape(_B, _S * _D)
